# baseline (speedup 1.0000x reference)
import functools

import jax
import jax.numpy as jnp
from jax import lax
from jax.experimental import pallas as pl
from jax.experimental.pallas import tpu as pltpu
from jax.experimental.pallas import tpu_sc as plsc

D = 128
TROWS = 128
KPAD = 8
C = 128
NC = 2
NS = 16
NW = NC * NS
NB = 7
STAGE = 8


def _proj_body(fm_ref, w_ref, b_ref, o_ref):
    o_ref[...] = (
        jnp.dot(fm_ref[...], w_ref[...], preferred_element_type=jnp.float32)
        + b_ref[...]
    )


def _build_table(fm_pad, w_pad, b_row):
    return pl.pallas_call(
        _proj_body,
        out_shape=jax.ShapeDtypeStruct((TROWS, D), jnp.float32),
    )(fm_pad, w_pad, b_row)


def _make_gather(n_rows):
    nchunk = n_rows // C
    nk = -(-nchunk // NW)
    nkp = -(-nk // (NB * STAGE)) * NB * STAGE
    ng = nkp // NB
    nchunk_pad = NW * nkp

    mesh = plsc.VectorSubcoreMesh(core_axis_name="c", subcore_axis_name="s")

    @functools.partial(
        pl.kernel,
        mesh=mesh,
        out_type=jax.ShapeDtypeStruct((n_rows, D), jnp.float32),
        scratch_types=[
            pltpu.VMEM((2 * STAGE * NB * C,), jnp.int32),
            pltpu.VMEM((NB, C, D), jnp.float32),
            pltpu.VMEM_SHARED((TROWS, D), jnp.float32),
        ]
        + [pltpu.SemaphoreType.DMA] * (1 + 2 * NB),
    )
    def gather(table_hbm, idxc_hbm, out_hbm, idx_v, rows_v, table_v, *sems):
        i_sem = sems[0]
        g_sem = sems[1 : 1 + NB]
        o_sem = sems[1 + NB : 1 + 2 * NB]
        wid = lax.axis_index("s") * NC + lax.axis_index("c")
        chunk_w0 = wid * nkp

        @pl.when(lax.axis_index("s") == 0)
        def _():
            pltpu.sync_copy(table_hbm, table_v)

        plsc.subcore_barrier()

        SZ = STAGE * NB * C

        pltpu.async_copy(
            idxc_hbm.at[pl.ds(chunk_w0 * C, SZ)], idx_v.at[pl.ds(0, SZ)], i_sem
        )

        def group(m, carry):
            chunk0 = chunk_w0 + m * NB
            s = lax.div(m, STAGE)
            half_off = lax.rem(s, 2) * SZ
            stage_off = lax.rem(m, STAGE) * NB * C

            @pl.when(lax.rem(m, STAGE) == 0)
            def _():
                pltpu.make_async_copy(
                    idxc_hbm.at[pl.ds(0, SZ)], idx_v.at[pl.ds(0, SZ)], i_sem
                ).wait()

                @pl.when(m + STAGE < ng)
                def _():
                    nxt_off = pl.multiple_of((1 - lax.rem(s, 2)) * SZ, C)
                    pltpu.async_copy(
                        idxc_hbm.at[pl.ds((chunk0 + STAGE * NB) * C, SZ)],
                        idx_v.at[pl.ds(nxt_off, SZ)],
                        i_sem,
                    )

            for b in range(NB):
                chunk = chunk0 + b
                ioff = pl.multiple_of(half_off + stage_off + b * C, C)
                prev_valid = (m > 0) & (chunk - NB < nchunk)

                @pl.when(prev_valid)
                def _():
                    pltpu.make_async_copy(
                        rows_v.at[b], out_hbm.at[pl.ds(0, C)], o_sem[b]
                    ).wait()

                @pl.when(chunk < nchunk)
                def _():
                    pltpu.async_copy(
                        table_v.at[idx_v.at[pl.ds(ioff, C)]],
                        rows_v.at[b],
                        g_sem[b],
                    )

            for b in range(NB):
                chunk = chunk0 + b
                ioff = pl.multiple_of(half_off + stage_off + b * C, C)

                @pl.when(chunk < nchunk)
                def _():
                    pltpu.make_async_copy(
                        table_v.at[idx_v.at[pl.ds(ioff, C)]],
                        rows_v.at[b],
                        g_sem[b],
                    ).wait()
                    pltpu.async_copy(
                        rows_v.at[b], out_hbm.at[pl.ds(chunk * C, C)], o_sem[b]
                    )

            return carry

        lax.fori_loop(0, ng, group, 0)
        for b in range(NB):
            chunk_last = chunk_w0 + (ng - 1) * NB + b

            @pl.when(chunk_last < nchunk)
            def _():
                pltpu.make_async_copy(
                    rows_v.at[b], out_hbm.at[pl.ds(0, C)], o_sem[b]
                ).wait()

    def run(table, idx):
        pad = nchunk_pad * C - n_rows
        idxc = jnp.pad(idx, (0, pad))
        return gather(table, idxc)

    return run


def kernel(src, feature_map, W, b):
    fm_pad = jnp.zeros((TROWS, KPAD), jnp.float32).at[:119, :3].set(feature_map)
    w_pad = jnp.zeros((KPAD, D), jnp.float32).at[:3].set(W)
    table = _build_table(fm_pad, w_pad, b.reshape(1, D).astype(jnp.float32))
    idx = src.astype(jnp.int32)
    return _make_gather(src.shape[0])(table, idx)


# speedup vs baseline: 1.0031x; 1.0031x over previous
import functools

import jax
import jax.numpy as jnp
from jax import lax
from jax.experimental import pallas as pl
from jax.experimental.pallas import tpu as pltpu
from jax.experimental.pallas import tpu_sc as plsc

D = 128
TROWS = 128
KPAD = 8
C = 128
NC = 2
NS = 16
NW = NC * NS
NB = 7
STAGE = 4


def _proj_body(fm_ref, w_ref, b_ref, o_ref):
    n = fm_ref.shape[0]
    o_ref[pl.ds(0, n), :] = (
        jnp.dot(fm_ref[...], w_ref[...], preferred_element_type=jnp.float32)
        + b_ref[...]
    )


def _build_table(fm, w, b_row):
    return pl.pallas_call(
        _proj_body,
        out_shape=jax.ShapeDtypeStruct((TROWS, D), jnp.float32),
    )(fm, w, b_row)


def _make_gather(n_rows):
    nchunk = n_rows // C
    nk = -(-nchunk // NW)
    nkp = -(-nk // (NB * STAGE)) * NB * STAGE
    ng = nkp // NB
    nchunk_pad = NW * nkp

    mesh = plsc.VectorSubcoreMesh(core_axis_name="c", subcore_axis_name="s")

    @functools.partial(
        pl.kernel,
        mesh=mesh,
        out_type=jax.ShapeDtypeStruct((n_rows, D), jnp.float32),
        scratch_types=[
            pltpu.VMEM((2 * STAGE * NB * C,), jnp.int32),
            pltpu.VMEM((NB, C, D), jnp.float32),
            pltpu.VMEM_SHARED((TROWS, D), jnp.float32),
        ]
        + [pltpu.SemaphoreType.DMA] * (1 + 2 * NB),
    )
    def gather(table_hbm, idxc_hbm, out_hbm, idx_v, rows_v, table_v, *sems):
        i_sem = sems[0]
        g_sem = sems[1 : 1 + NB]
        o_sem = sems[1 + NB : 1 + 2 * NB]
        wid = lax.axis_index("s") * NC + lax.axis_index("c")
        chunk_w0 = wid * nkp

        @pl.when(lax.axis_index("s") == 0)
        def _():
            pltpu.sync_copy(table_hbm, table_v)

        plsc.subcore_barrier()

        SZ = STAGE * NB * C

        pltpu.async_copy(
            idxc_hbm.at[pl.ds(chunk_w0 * C, SZ)], idx_v.at[pl.ds(0, SZ)], i_sem
        )

        def group(m, carry):
            chunk0 = chunk_w0 + m * NB
            s = lax.div(m, STAGE)
            half_off = lax.rem(s, 2) * SZ
            stage_off = lax.rem(m, STAGE) * NB * C

            @pl.when(lax.rem(m, STAGE) == 0)
            def _():
                pltpu.make_async_copy(
                    idxc_hbm.at[pl.ds(0, SZ)], idx_v.at[pl.ds(0, SZ)], i_sem
                ).wait()

                @pl.when(m + STAGE < ng)
                def _():
                    nxt_off = pl.multiple_of((1 - lax.rem(s, 2)) * SZ, C)
                    pltpu.async_copy(
                        idxc_hbm.at[pl.ds((chunk0 + STAGE * NB) * C, SZ)],
                        idx_v.at[pl.ds(nxt_off, SZ)],
                        i_sem,
                    )

            for b in range(NB):
                chunk = chunk0 + b
                ioff = pl.multiple_of(half_off + stage_off + b * C, C)
                prev_valid = (m > 0) & (chunk - NB < nchunk)

                @pl.when(prev_valid)
                def _():
                    pltpu.make_async_copy(
                        rows_v.at[b], out_hbm.at[pl.ds(0, C)], o_sem[b]
                    ).wait()

                @pl.when(chunk < nchunk)
                def _():
                    pltpu.async_copy(
                        table_v.at[idx_v.at[pl.ds(ioff, C)]],
                        rows_v.at[b],
                        g_sem[b],
                    )

            for b in range(NB):
                chunk = chunk0 + b
                ioff = pl.multiple_of(half_off + stage_off + b * C, C)

                @pl.when(chunk < nchunk)
                def _():
                    pltpu.make_async_copy(
                        table_v.at[idx_v.at[pl.ds(ioff, C)]],
                        rows_v.at[b],
                        g_sem[b],
                    ).wait()
                    pltpu.async_copy(
                        rows_v.at[b], out_hbm.at[pl.ds(chunk * C, C)], o_sem[b]
                    )

            return carry

        lax.fori_loop(0, ng, group, 0)
        for b in range(NB):
            chunk_last = chunk_w0 + (ng - 1) * NB + b

            @pl.when(chunk_last < nchunk)
            def _():
                pltpu.make_async_copy(
                    rows_v.at[b], out_hbm.at[pl.ds(0, C)], o_sem[b]
                ).wait()

    def run(table, idx):
        pad = nchunk_pad * C - n_rows
        idxc = jnp.pad(idx, (0, pad))
        return gather(table, idxc)

    return run


def kernel(src, feature_map, W, b):
    table = _build_table(
        feature_map.astype(jnp.float32),
        W.astype(jnp.float32),
        b.reshape(1, D).astype(jnp.float32),
    )
    idx = src.astype(jnp.int32)
    return _make_gather(src.shape[0])(table, idx)
